# baseline (speedup 1.0000x reference)
import functools

import jax
import jax.numpy as jnp
from jax import lax
from jax.experimental import pallas as pl
from jax.experimental.pallas import tpu as pltpu
from jax.experimental.pallas import tpu_sc as plsc

_NEG = float("-inf")


def _chunk_top(x, vi, beam):
    big = jnp.int32(2**30)
    cv, ci = [], []
    for k in range(beam):
        m = jnp.max(x)
        g = jnp.min(jnp.where(x == m, vi, big))
        cv.append(m)
        ci.append(g)
        if k + 1 < beam:
            x = jnp.where(vi == g, _NEG, x)
    return cv, ci


def _tc_body(lg_ref, *refs, n_kv, vocab, beam, cl, n_slots):
    kv_in = refs[:n_kv]
    probs_ref = refs[n_kv]
    idx_ref = refs[n_kv + 1]
    kv_out = refs[n_kv + 2:n_kv + 2 + n_kv]
    slots, in_sems, out_sems = refs[n_kv + 2 + n_kv:]

    def in_copy(j):
        return pltpu.make_async_copy(kv_in[j].at[0], slots.at[j % n_slots],
                                     in_sems.at[j % n_slots])

    def out_copy(j, b):
        return pltpu.make_async_copy(slots.at[j % n_slots], kv_out[j].at[b],
                                     out_sems.at[j % n_slots, b])

    rows, cols = 8, vocab // 8
    n_full = cols // cl
    tail = cols - n_full * cl
    n_chunks = n_full + (1 if tail else 0)
    cand_v, cand_i = [], []
    cms, css = [], []

    def logits_chunk(j):
        if j >= n_chunks:
            return
        off = j * cl
        width = cl if j < n_full else tail
        x = lg_ref[:, pl.ds(off, width)]
        vi = (jax.lax.broadcasted_iota(jnp.int32, (rows, width), 0) * cols
              + jax.lax.broadcasted_iota(jnp.int32, (rows, width), 1) + off)
        cv, ci = _chunk_top(x, vi, beam)
        cand_v.extend(cv)
        cand_i.extend(ci)
        cms.append(cv[0])
        css.append(jnp.sum(jnp.exp(x - cv[0])))

    for j in range(min(n_slots - 1, n_kv)):
        in_copy(j).start()

    chunks_per_layer = -(-n_chunks // n_kv)
    for j in range(n_kv):
        in_copy(j).wait()
        for b in range(beam):
            out_copy(j, b).start()
        nxt = j + n_slots - 1
        if nxt < n_kv:
            if nxt - n_slots >= 0:
                for b in range(beam):
                    out_copy(nxt - n_slots, b).wait()
            in_copy(nxt).start()
        for q in range(chunks_per_layer):
            logits_chunk(j * chunks_per_layer + q)

    for j in range(max(0, n_kv - n_slots), n_kv):
        for b in range(beam):
            out_copy(j, b).wait()

    big = jnp.int32(2**30)
    nc = len(cand_v)
    lane_c = jax.lax.broadcasted_iota(jnp.int32, (1, nc), 1)
    candv = jnp.zeros((1, nc), jnp.float32)
    candi = jnp.zeros((1, nc), jnp.int32)
    for k in range(nc):
        candv = jnp.where(lane_c == k, cand_v[k], candv)
        candi = jnp.where(lane_c == k, cand_i[k], candi)

    mg = cms[0]
    for c in cms[1:]:
        mg = jnp.maximum(mg, c)
    sg = css[0] * jnp.exp(cms[0] - mg)
    for c, s in zip(cms[1:], css[1:]):
        sg = sg + s * jnp.exp(c - mg)
    lse = jnp.log(sg) + mg

    lane_b = jax.lax.broadcasted_iota(jnp.int32, (1, beam), 1)
    pv = jnp.zeros((1, beam), jnp.float32)
    iv = jnp.zeros((1, beam), jnp.int32)
    for k in range(beam):
        m = jnp.max(candv)
        g = jnp.min(jnp.where(candv == m, candi, big))
        pv = jnp.where(lane_b == k, m - lse, pv)
        iv = jnp.where(lane_b == k, g, iv)
        candv = jnp.where(candi == g, _NEG, candv)
    probs_ref[...] = pv
    idx_ref[...] = iv


def _sc_copy_body(*refs, n_kv, beam, seq_half):
    kv_in = refs[:n_kv]
    kv_out = refs[n_kv:2 * n_kv]
    buf0, buf1, in_sem0, in_sem1, out_sem0, out_sem1 = refs[2 * n_kv:]
    bufs = (buf0, buf1)
    in_sems = (in_sem0, in_sem1)
    out_sems = (out_sem0, out_sem1)

    c = lax.axis_index("c")
    s = lax.axis_index("s")
    sub = seq_half // 2
    n_units = 2 * n_kv

    def in_copy(u):
        j, h = u // 2, u % 2
        soff = c * seq_half + h * sub
        return pltpu.make_async_copy(
            kv_in[j].at[0, s, pl.ds(soff, sub), :],
            bufs[u % 2], in_sems[u % 2])

    def out_copy(u, b):
        j, h = u // 2, u % 2
        soff = c * seq_half + h * sub
        return pltpu.make_async_copy(
            bufs[u % 2], kv_out[j].at[b, s, pl.ds(soff, sub), :],
            out_sems[u % 2])

    in_copy(0).start()
    for u in range(n_units):
        if u + 1 < n_units:
            if u - 1 >= 0:
                for b in range(beam):
                    out_copy(u - 1, b).wait()
            in_copy(u + 1).start()
        in_copy(u).wait()
        for b in range(beam):
            out_copy(u, b).start()
    for u in (n_units - 2, n_units - 1):
        for b in range(beam):
            out_copy(u, b).wait()


def kernel(kv_0, kv_1, kv_2, kv_3, kv_4, kv_5, kv_6, kv_7, kv_8, kv_9,
           kv_10, kv_11, logits, save_id, beam_size):
    kvs = [kv_0, kv_1, kv_2, kv_3, kv_4, kv_5, kv_6, kv_7, kv_8, kv_9,
           kv_10, kv_11]
    n_kv = len(kvs)
    beam = save_id.shape[0]
    kv_shape = kvs[0].shape
    vocab = logits.shape[-1]
    seq_half = kv_shape[2] // 2

    n_tc = 4
    tc_kvs, sc_kvs = kvs[:n_tc], kvs[n_tc:]
    n_sc = n_kv - n_tc

    mesh = plsc.VectorSubcoreMesh(core_axis_name="c", subcore_axis_name="s")
    sc_copy = functools.partial(
        pl.kernel,
        mesh=mesh,
        out_type=[jax.ShapeDtypeStruct((beam,) + kv_shape[1:],
                                       jnp.float32)] * n_sc,
        scratch_types=[pltpu.VMEM((seq_half // 2, kv_shape[3]), jnp.float32),
                       pltpu.VMEM((seq_half // 2, kv_shape[3]), jnp.float32),
                       pltpu.SemaphoreType.DMA,
                       pltpu.SemaphoreType.DMA,
                       pltpu.SemaphoreType.DMA,
                       pltpu.SemaphoreType.DMA],
    )(functools.partial(_sc_copy_body, n_kv=n_sc, beam=beam,
                        seq_half=seq_half))
    sc_outs = list(sc_copy(*sc_kvs))

    cl = 10368
    n_slots = 2
    body = functools.partial(_tc_body, n_kv=n_tc, vocab=vocab, beam=beam,
                             cl=cl, n_slots=n_slots)
    in_specs = [pl.BlockSpec(memory_space=pltpu.MemorySpace.VMEM)]
    in_specs += [pl.BlockSpec(memory_space=pl.ANY)] * n_tc
    out_specs = [pl.BlockSpec(memory_space=pltpu.MemorySpace.VMEM),
                 pl.BlockSpec(memory_space=pltpu.MemorySpace.VMEM)]
    out_specs += [pl.BlockSpec(memory_space=pl.ANY)] * n_tc
    out_shape = [jax.ShapeDtypeStruct((1, beam), jnp.float32),
                 jax.ShapeDtypeStruct((1, beam), jnp.int32)]
    out_shape += [jax.ShapeDtypeStruct((beam,) + kv_shape[1:],
                                       jnp.float32)] * n_tc

    outs = pl.pallas_call(
        body,
        in_specs=in_specs,
        out_specs=out_specs,
        out_shape=out_shape,
        scratch_shapes=[pltpu.VMEM((n_slots,) + kv_shape[1:], jnp.float32),
                        pltpu.SemaphoreType.DMA((n_slots,)),
                        pltpu.SemaphoreType.DMA((n_slots, beam))],
    )(logits.reshape(8, vocab // 8), *tc_kvs)

    probs, idx = outs[0], outs[1]
    kv_outs = list(outs[2:]) + sc_outs

    idx_t = idx.reshape(beam, 1)
    save_id_out = jnp.concatenate([save_id, idx_t], axis=-1)
    probs_t = probs.reshape(beam, 1)
    bz = jnp.asarray(beam_size, jnp.int32) - jnp.int32(beam)
    max_idx = idx_t[0] + bz
    return (*kv_outs, idx_t, save_id_out, probs_t, max_idx)


# speedup vs baseline: 1.0191x; 1.0191x over previous
import functools

import jax
import jax.numpy as jnp
from jax import lax
from jax.experimental import pallas as pl
from jax.experimental.pallas import tpu as pltpu
from jax.experimental.pallas import tpu_sc as plsc

_NEG = float("-inf")


def _chunk_top(x, vi, beam):
    big = jnp.int32(2**30)
    cv, ci = [], []
    for k in range(beam):
        m = jnp.max(x)
        g = jnp.min(jnp.where(x == m, vi, big))
        cv.append(m)
        ci.append(g)
        if k + 1 < beam:
            x = jnp.where(vi == g, _NEG, x)
    return cv, ci


def _tc_body(lg_ref, *refs, n_kv, vocab, beam, cl, n_slots):
    kv_in = refs[:n_kv]
    probs_ref = refs[n_kv]
    idx_ref = refs[n_kv + 1]
    kv_out = refs[n_kv + 2:n_kv + 2 + n_kv]
    slots, in_sems, out_sems = refs[n_kv + 2 + n_kv:]

    def in_copy(j):
        return pltpu.make_async_copy(kv_in[j].at[0], slots.at[j % n_slots],
                                     in_sems.at[j % n_slots])

    def out_copy(j, b):
        return pltpu.make_async_copy(slots.at[j % n_slots], kv_out[j].at[b],
                                     out_sems.at[j % n_slots, b])

    sub = cl // 8
    n_full = vocab // cl
    tail = vocab - n_full * cl
    n_chunks = n_full + (1 if tail else 0)
    cand_v, cand_i = [], []
    cms, css = [], []

    def logits_chunk(j):
        if j >= n_chunks:
            return
        off = j * cl
        if j < n_full:
            x = jnp.concatenate(
                [lg_ref[:, pl.ds(off + r * sub, sub)] for r in range(8)],
                axis=0)
            vi = (jax.lax.broadcasted_iota(jnp.int32, (8, sub), 0) * sub
                  + jax.lax.broadcasted_iota(jnp.int32, (8, sub), 1) + off)
        else:
            x = lg_ref[:, pl.ds(off, tail)]
            vi = jax.lax.broadcasted_iota(jnp.int32, (1, tail), 1) + off
        cv, ci = _chunk_top(x, vi, beam)
        cand_v.extend(cv)
        cand_i.extend(ci)
        cms.append(cv[0])
        css.append(jnp.sum(jnp.exp(x - cv[0])))

    for j in range(min(n_slots - 1, n_kv)):
        in_copy(j).start()

    chunks_per_layer = -(-n_chunks // n_kv)
    for j in range(n_kv):
        in_copy(j).wait()
        for b in range(beam):
            out_copy(j, b).start()
        nxt = j + n_slots - 1
        if nxt < n_kv:
            if nxt - n_slots >= 0:
                for b in range(beam):
                    out_copy(nxt - n_slots, b).wait()
            in_copy(nxt).start()
        for q in range(chunks_per_layer):
            logits_chunk(j * chunks_per_layer + q)

    for j in range(max(0, n_kv - n_slots), n_kv):
        for b in range(beam):
            out_copy(j, b).wait()

    big = jnp.int32(2**30)
    nc = len(cand_v)
    lane_c = jax.lax.broadcasted_iota(jnp.int32, (1, nc), 1)
    candv = jnp.zeros((1, nc), jnp.float32)
    candi = jnp.zeros((1, nc), jnp.int32)
    for k in range(nc):
        candv = jnp.where(lane_c == k, cand_v[k], candv)
        candi = jnp.where(lane_c == k, cand_i[k], candi)

    mg = cms[0]
    for c in cms[1:]:
        mg = jnp.maximum(mg, c)
    sg = css[0] * jnp.exp(cms[0] - mg)
    for c, s in zip(cms[1:], css[1:]):
        sg = sg + s * jnp.exp(c - mg)
    lse = jnp.log(sg) + mg

    lane_b = jax.lax.broadcasted_iota(jnp.int32, (1, beam), 1)
    pv = jnp.zeros((1, beam), jnp.float32)
    iv = jnp.zeros((1, beam), jnp.int32)
    for k in range(beam):
        m = jnp.max(candv)
        g = jnp.min(jnp.where(candv == m, candi, big))
        pv = jnp.where(lane_b == k, m - lse, pv)
        iv = jnp.where(lane_b == k, g, iv)
        candv = jnp.where(candi == g, _NEG, candv)
    probs_ref[...] = pv
    idx_ref[...] = iv


def _sc_copy_body(*refs, n_kv, beam, seq_half):
    kv_in = refs[:n_kv]
    kv_out = refs[n_kv:2 * n_kv]
    buf0, buf1, in_sem0, in_sem1, out_sem0, out_sem1 = refs[2 * n_kv:]
    bufs = (buf0, buf1)
    in_sems = (in_sem0, in_sem1)
    out_sems = (out_sem0, out_sem1)

    c = lax.axis_index("c")
    s = lax.axis_index("s")
    sub = seq_half // 2
    n_units = 2 * n_kv

    def in_copy(u):
        j, h = u // 2, u % 2
        soff = c * seq_half + h * sub
        return pltpu.make_async_copy(
            kv_in[j].at[0, s, pl.ds(soff, sub), :],
            bufs[u % 2], in_sems[u % 2])

    def out_copy(u, b):
        j, h = u // 2, u % 2
        soff = c * seq_half + h * sub
        return pltpu.make_async_copy(
            bufs[u % 2], kv_out[j].at[b, s, pl.ds(soff, sub), :],
            out_sems[u % 2])

    in_copy(0).start()
    for u in range(n_units):
        if u + 1 < n_units:
            if u - 1 >= 0:
                for b in range(beam):
                    out_copy(u - 1, b).wait()
            in_copy(u + 1).start()
        in_copy(u).wait()
        for b in range(beam):
            out_copy(u, b).start()
    for u in (n_units - 2, n_units - 1):
        for b in range(beam):
            out_copy(u, b).wait()


def kernel(kv_0, kv_1, kv_2, kv_3, kv_4, kv_5, kv_6, kv_7, kv_8, kv_9,
           kv_10, kv_11, logits, save_id, beam_size):
    kvs = [kv_0, kv_1, kv_2, kv_3, kv_4, kv_5, kv_6, kv_7, kv_8, kv_9,
           kv_10, kv_11]
    n_kv = len(kvs)
    beam = save_id.shape[0]
    kv_shape = kvs[0].shape
    vocab = logits.shape[-1]
    seq_half = kv_shape[2] // 2

    n_tc = 4
    tc_kvs, sc_kvs = kvs[:n_tc], kvs[n_tc:]
    n_sc = n_kv - n_tc

    mesh = plsc.VectorSubcoreMesh(core_axis_name="c", subcore_axis_name="s")
    sc_copy = functools.partial(
        pl.kernel,
        mesh=mesh,
        out_type=[jax.ShapeDtypeStruct((beam,) + kv_shape[1:],
                                       jnp.float32)] * n_sc,
        scratch_types=[pltpu.VMEM((seq_half // 2, kv_shape[3]), jnp.float32),
                       pltpu.VMEM((seq_half // 2, kv_shape[3]), jnp.float32),
                       pltpu.SemaphoreType.DMA,
                       pltpu.SemaphoreType.DMA,
                       pltpu.SemaphoreType.DMA,
                       pltpu.SemaphoreType.DMA],
    )(functools.partial(_sc_copy_body, n_kv=n_sc, beam=beam,
                        seq_half=seq_half))
    sc_outs = list(sc_copy(*sc_kvs))

    cl = 82944
    n_slots = 2
    body = functools.partial(_tc_body, n_kv=n_tc, vocab=vocab, beam=beam,
                             cl=cl, n_slots=n_slots)
    in_specs = [pl.BlockSpec(memory_space=pltpu.MemorySpace.VMEM)]
    in_specs += [pl.BlockSpec(memory_space=pl.ANY)] * n_tc
    out_specs = [pl.BlockSpec(memory_space=pltpu.MemorySpace.VMEM),
                 pl.BlockSpec(memory_space=pltpu.MemorySpace.VMEM)]
    out_specs += [pl.BlockSpec(memory_space=pl.ANY)] * n_tc
    out_shape = [jax.ShapeDtypeStruct((1, beam), jnp.float32),
                 jax.ShapeDtypeStruct((1, beam), jnp.int32)]
    out_shape += [jax.ShapeDtypeStruct((beam,) + kv_shape[1:],
                                       jnp.float32)] * n_tc

    outs = pl.pallas_call(
        body,
        in_specs=in_specs,
        out_specs=out_specs,
        out_shape=out_shape,
        scratch_shapes=[pltpu.VMEM((n_slots,) + kv_shape[1:], jnp.float32),
                        pltpu.SemaphoreType.DMA((n_slots,)),
                        pltpu.SemaphoreType.DMA((n_slots, beam))],
    )(logits, *tc_kvs)

    probs, idx = outs[0], outs[1]
    kv_outs = list(outs[2:]) + sc_outs

    idx_t = idx.reshape(beam, 1)
    save_id_out = jnp.concatenate([save_id, idx_t], axis=-1)
    probs_t = probs.reshape(beam, 1)
    bz = jnp.asarray(beam_size, jnp.int32) - jnp.int32(beam)
    max_idx = idx_t[0] + bz
    return (*kv_outs, idx_t, save_id_out, probs_t, max_idx)
